# baseline (speedup 1.0000x reference)
import jax
import jax.numpy as jnp
from jax import lax
from jax.experimental import pallas as pl
from jax.experimental.pallas import tpu as pltpu

EMB = 32
QS = 64
SH = 64
L = 50
T = 32768
BLK = 2048
NB = T // BLK


def _dotT(a, b):
    return lax.dot_general(a, b, (((1,), (1,)), ((), ())),
                           preferred_element_type=jnp.float32,
                           precision=lax.Precision.HIGHEST)


def _gru(gi, gh, h):
    H = h.shape[1]
    r = jax.nn.sigmoid(gi[:, :H] + gh[:, :H])
    z = jax.nn.sigmoid(gi[:, H:2 * H] + gh[:, H:2 * H])
    n = jnp.tanh(gi[:, 2 * H:] + r * gh[:, 2 * H:])
    return (1.0 - z) * n + z * h


def _ques_kernel(question_ref, score_ref, hlast_ref, emb_hbm,
                 Wih_f, Whh_f, bih_f, bhh_f,
                 Wih_b, Whh_b, bih_b, bhh_b,
                 gWih, gWhh, gbih, gbhh,
                 q_out, hnew_out,
                 x_scr, gif_scr, gib_scr, sem):
    def _cp(j):
        return pltpu.make_async_copy(
            emb_hbm.at[pl.ds(question_ref[j], 1), :],
            x_scr.at[pl.ds(j, 1), :], sem)

    def _start(j, c):
        _cp(j).start()
        return c

    def _wait(j, c):
        _cp(j).wait()
        return c

    lax.fori_loop(0, L, _start, 0)
    lax.fori_loop(0, L, _wait, 0)

    x = x_scr[...]
    gif_scr[...] = _dotT(x, Wih_f[...]) + bih_f[...]
    gib_scr[...] = _dotT(x, Wih_b[...]) + bih_b[...]

    def step(t, carry):
        h_f, h_b, mf, mb = carry
        gif = gif_scr[pl.ds(t, 1), :]
        ghf = _dotT(h_f, Whh_f[...]) + bhh_f[...]
        h_f = _gru(gif, ghf, h_f)
        gib = gib_scr[pl.ds(L - 1 - t, 1), :]
        ghb = _dotT(h_b, Whh_b[...]) + bhh_b[...]
        h_b = _gru(gib, ghb, h_b)
        return (h_f, h_b, jnp.maximum(mf, h_f), jnp.maximum(mb, h_b))

    zeros = jnp.zeros((1, EMB), jnp.float32)
    ninf = jnp.full((1, EMB), -jnp.inf, jnp.float32)
    _, _, mf, mb = lax.fori_loop(0, L, step, (zeros, zeros, ninf, ninf))
    q = jnp.concatenate([mf, mb], axis=1)
    q_out[...] = q

    s = score_ref[0]
    pos = (s >= 0.5).astype(jnp.float32)
    x_in = jnp.concatenate([q * pos, q * (1.0 - pos)], axis=1)
    gi = _dotT(x_in, gWih[...]) + gbih[...]
    gh = _dotT(hlast_ref[...], gWhh[...]) + gbhh[...]
    hnew_out[...] = _gru(gi, gh, hlast_ref[...])


def _stream_kernel(qh_ref, hs_any, q_ref, hnew_ref, sW_ref, sb_ref,
                   qn_out, hn_any, pred_out,
                   run_max, gidx, row_scr, sem_hs, sem_hnew, sem_row):
    i = pl.program_id(0)

    @pl.when(i == 0)
    def _():
        run_max[0] = -jnp.inf
        pltpu.make_async_copy(hs_any, hn_any.at[pl.ds(0, T), :, :],
                              sem_hs).start()
        pltpu.make_async_copy(hnew_ref, hn_any.at[pl.ds(T, 1), 0, :],
                              sem_hnew).start()

    @pl.when(i < NB)
    def _():
        blk = qh_ref[...]
        qn_out[...] = blk
        alpha = jnp.sum(blk * q_ref[...], axis=1, keepdims=True)
        m = jnp.max(alpha)

        @pl.when(m > run_max[0])
        def _():
            run_max[0] = m
            rows = lax.broadcasted_iota(jnp.int32, (BLK, 1), 0)
            a = jnp.min(jnp.where(alpha >= m, rows, BLK))
            gidx[0] = i * BLK + a

    @pl.when(i == NB)
    def _():
        qn_out[pl.ds(0, 1), :] = q_ref[...]
        pltpu.make_async_copy(hs_any.at[pl.ds(gidx[0], 1), 0, :],
                              row_scr, sem_row).start()
        pltpu.make_async_copy(hs_any.at[pl.ds(gidx[0], 1), 0, :],
                              row_scr, sem_row).wait()
        pred_out[...] = (jnp.sum(q_ref[...] * sW_ref[:, :QS],
                                 axis=1, keepdims=True)
                         + jnp.sum(row_scr[...] * sW_ref[:, QS:],
                                   axis=1, keepdims=True)
                         + sb_ref[0])
        pltpu.make_async_copy(hs_any, hn_any.at[pl.ds(0, T), :, :],
                              sem_hs).wait()
        pltpu.make_async_copy(hnew_ref, hn_any.at[pl.ds(T, 1), 0, :],
                              sem_hnew).wait()


def kernel(question, score, questions_hist, hs_hist, emb,
           qWih_f, qWhh_f, qbih_f, qbhh_f,
           qWih_b, qWhh_b, qbih_b, qbhh_b,
           sW, sb, gWih, gWhh, gbih, gbhh):
    question = question.astype(jnp.int32)
    hlast = lax.slice(hs_hist, (T - 1, 0, 0), (T, 1, SH)).reshape(1, SH)
    f32 = jnp.float32

    q, hnew = pl.pallas_call(
        _ques_kernel,
        out_shape=[jax.ShapeDtypeStruct((1, QS), f32),
                   jax.ShapeDtypeStruct((1, SH), f32)],
        in_specs=[
            pl.BlockSpec(memory_space=pltpu.MemorySpace.SMEM),
            pl.BlockSpec(memory_space=pltpu.MemorySpace.SMEM),
            pl.BlockSpec(memory_space=pltpu.MemorySpace.VMEM),
            pl.BlockSpec(memory_space=pltpu.MemorySpace.HBM),
        ] + [pl.BlockSpec(memory_space=pltpu.MemorySpace.VMEM)] * 12,
        scratch_shapes=[pltpu.VMEM((L, EMB), f32),
                        pltpu.VMEM((L, 3 * EMB), f32),
                        pltpu.VMEM((L, 3 * EMB), f32),
                        pltpu.SemaphoreType.DMA],
    )(question, score.astype(f32), hlast, emb,
      qWih_f, qWhh_f, qbih_f.reshape(1, -1), qbhh_f.reshape(1, -1),
      qWih_b, qWhh_b, qbih_b.reshape(1, -1), qbhh_b.reshape(1, -1),
      gWih, gWhh, gbih.reshape(1, -1), gbhh.reshape(1, -1))

    qn, hn, pred = pl.pallas_call(
        _stream_kernel,
        grid=(NB + 1,),
        in_specs=[
            pl.BlockSpec((BLK, QS), lambda i: (jnp.minimum(i, NB - 1), 0)),
            pl.BlockSpec(memory_space=pltpu.MemorySpace.HBM),
            pl.BlockSpec((1, QS), lambda i: (0, 0)),
            pl.BlockSpec((1, SH), lambda i: (0, 0)),
            pl.BlockSpec((1, QS + SH), lambda i: (0, 0)),
            pl.BlockSpec(memory_space=pltpu.MemorySpace.SMEM),
        ],
        out_specs=[
            pl.BlockSpec((BLK, QS), lambda i: (i, 0)),
            pl.BlockSpec(memory_space=pltpu.MemorySpace.HBM),
            pl.BlockSpec((1, 1), lambda i: (0, 0)),
        ],
        out_shape=[
            jax.ShapeDtypeStruct((T + 1, QS), f32),
            jax.ShapeDtypeStruct((T + 1, 1, SH), f32),
            jax.ShapeDtypeStruct((1, 1), f32),
        ],
        scratch_shapes=[pltpu.SMEM((1,), f32), pltpu.SMEM((1,), jnp.int32),
                        pltpu.VMEM((1, SH), f32),
                        pltpu.SemaphoreType.DMA, pltpu.SemaphoreType.DMA,
                        pltpu.SemaphoreType.DMA],
    )(questions_hist, hs_hist, q, hnew, sW, sb.astype(f32))

    return pred, qn, hn


# speedup vs baseline: 12.3545x; 12.3545x over previous
import jax
import jax.numpy as jnp
from jax import lax
from jax.experimental import pallas as pl
from jax.experimental.pallas import tpu as pltpu

EMB = 32
QS = 64
SH = 64
L = 50
T = 32768
BLK = 2048
NB = T // BLK


def _dot(a, b, ca, cb):
    return lax.dot_general(a, b, (((ca,), (cb,)), ((), ())),
                           preferred_element_type=jnp.float32,
                           precision=lax.Precision.HIGHEST)


def _gru(gi, gh, h):
    H = h.shape[1]
    r = jax.nn.sigmoid(gi[:, :H] + gh[:, :H])
    z = jax.nn.sigmoid(gi[:, H:2 * H] + gh[:, H:2 * H])
    n = jnp.tanh(gi[:, 2 * H:] + r * gh[:, 2 * H:])
    return (1.0 - z) * n + z * h


def _ques_kernel(question_ref, score_ref, hlast_ref, embT_hbm,
                 WihT_f, WhhT_f, bih_f, bhh_f,
                 WihT_b, WhhT_b, bih_b, bhh_b,
                 gWih, gWhhT, gbih, gbhh,
                 q_out, hnew_out,
                 tiles_scr, gif_scr, gib_scr, sem):
    def _cp(j):
        base = pl.multiple_of((question_ref[j] // 128) * 128, 128)
        return pltpu.make_async_copy(
            embT_hbm.at[:, pl.ds(base, 128)],
            tiles_scr.at[:, pl.ds(j * 128, 128)], sem)

    for j in range(L):
        _cp(j).start()
    for j in range(L):
        _cp(j).wait()

    laneL = lax.broadcasted_iota(jnp.int32, (1, L), 1)
    r_vec = jnp.zeros((1, L), jnp.int32)
    for j in range(L):
        r_vec = jnp.where(laneL == j, question_ref[j] % 128, r_vec)
    k_iota = lax.broadcasted_iota(jnp.int32, (L * 128, L), 0)
    j_iota = lax.broadcasted_iota(jnp.int32, (L * 128, L), 1)
    sel = ((k_iota // 128 == j_iota) & (k_iota % 128 == r_vec)
           ).astype(jnp.float32)
    x_cols = _dot(tiles_scr[...], sel, 1, 0)
    gif_scr[...] = _dot(x_cols, WihT_f[...], 0, 0) + bih_f[...]
    gib_scr[...] = _dot(x_cols, WihT_b[...], 0, 0) + bih_b[...]

    def step(t, carry):
        h_f, h_b, mf, mb = carry
        gif = gif_scr[pl.ds(t, 1), :]
        ghf = _dot(h_f, WhhT_f[...], 1, 0) + bhh_f[...]
        h_f = _gru(gif, ghf, h_f)
        gib = gib_scr[pl.ds(L - 1 - t, 1), :]
        ghb = _dot(h_b, WhhT_b[...], 1, 0) + bhh_b[...]
        h_b = _gru(gib, ghb, h_b)
        return (h_f, h_b, jnp.maximum(mf, h_f), jnp.maximum(mb, h_b))

    zeros = jnp.zeros((1, EMB), jnp.float32)
    ninf = jnp.full((1, EMB), -jnp.inf, jnp.float32)
    _, _, mf, mb = lax.fori_loop(0, L, step, (zeros, zeros, ninf, ninf))
    q = jnp.concatenate([mf, mb], axis=1)
    q_out[...] = q

    s = score_ref[0]
    pos = (s >= 0.5).astype(jnp.float32)
    x_in = jnp.concatenate([q * pos, q * (1.0 - pos)], axis=1)
    gi = _dot(x_in, gWih[...], 1, 1) + gbih[...]
    gh = _dot(hlast_ref[...], gWhhT[...], 1, 0) + gbhh[...]
    hnew_out[...] = _gru(gi, gh, hlast_ref[...])


def _row_to_col(row):
    n = row.shape[1]
    sub = lax.broadcasted_iota(jnp.int32, (n, n), 0)
    lanes = lax.broadcasted_iota(jnp.int32, (n, n), 1)
    diag = (sub == lanes).astype(jnp.float32)
    return jnp.sum(row * diag, axis=1, keepdims=True)


def _stream_kernel(qhT_ref, hsT_ref, q_ref, hnew_ref, sW_ref, sb_ref,
                   qnT_out, hnT_out, pred_out, run_max, run_col):
    i = pl.program_id(0)

    @pl.when(i == 0)
    def _():
        run_max[0] = -jnp.inf

    @pl.when(i < NB)
    def _():
        blk = qhT_ref[...]
        qnT_out[...] = blk
        hs_blk = hsT_ref[...]
        hnT_out[...] = hs_blk
        alpha = _dot(q_ref[...], blk, 1, 0)
        m = jnp.max(alpha)

        @pl.when(m > run_max[0])
        def _():
            run_max[0] = m
            lanes = lax.broadcasted_iota(jnp.int32, (1, BLK), 1)
            a = jnp.min(jnp.where(alpha >= m, lanes, BLK))
            onehot = (lanes == a).astype(jnp.float32)
            run_col[...] = jnp.sum(hs_blk * onehot, axis=1, keepdims=True)

    @pl.when(i == NB)
    def _():
        qnT_out[:, pl.ds(0, 1)] = _row_to_col(q_ref[...])
        hnT_out[:, pl.ds(0, 1)] = _row_to_col(hnew_ref[...])
        t1 = jnp.sum(q_ref[...] * sW_ref[:, :QS])
        t2 = jnp.sum(run_col[...] * _row_to_col(sW_ref[:, QS:]))
        pred_out[...] = jnp.zeros((1, 1), jnp.float32) + t1 + t2 + sb_ref[0]


def kernel(question, score, questions_hist, hs_hist, emb,
           qWih_f, qWhh_f, qbih_f, qbhh_f,
           qWih_b, qWhh_b, qbih_b, qbhh_b,
           sW, sb, gWih, gWhh, gbih, gbhh):
    question = question.astype(jnp.int32)
    f32 = jnp.float32

    qhT = questions_hist.T
    hsT = jnp.transpose(hs_hist, (1, 2, 0)).reshape(SH, T)
    embT = emb.T
    hlast = hs_hist[T - 1].reshape(1, SH)

    q, hnew = pl.pallas_call(
        _ques_kernel,
        out_shape=[jax.ShapeDtypeStruct((1, QS), f32),
                   jax.ShapeDtypeStruct((1, SH), f32)],
        in_specs=[
            pl.BlockSpec(memory_space=pltpu.MemorySpace.SMEM),
            pl.BlockSpec(memory_space=pltpu.MemorySpace.SMEM),
            pl.BlockSpec(memory_space=pltpu.MemorySpace.VMEM),
            pl.BlockSpec(memory_space=pltpu.MemorySpace.HBM),
        ] + [pl.BlockSpec(memory_space=pltpu.MemorySpace.VMEM)] * 12,
        scratch_shapes=[pltpu.VMEM((EMB, L * 128), f32),
                        pltpu.VMEM((L, 3 * EMB), f32),
                        pltpu.VMEM((L, 3 * EMB), f32),
                        pltpu.SemaphoreType.DMA],
    )(question, score.astype(f32), hlast, embT,
      qWih_f.T, qWhh_f.T, qbih_f.reshape(1, -1), qbhh_f.reshape(1, -1),
      qWih_b.T, qWhh_b.T, qbih_b.reshape(1, -1), qbhh_b.reshape(1, -1),
      gWih, gWhh.T, gbih.reshape(1, -1), gbhh.reshape(1, -1))

    qnT, hnT, pred = pl.pallas_call(
        _stream_kernel,
        grid=(NB + 1,),
        in_specs=[
            pl.BlockSpec((QS, BLK), lambda i: (0, jnp.minimum(i, NB - 1))),
            pl.BlockSpec((SH, BLK), lambda i: (0, jnp.minimum(i, NB - 1))),
            pl.BlockSpec((1, QS), lambda i: (0, 0)),
            pl.BlockSpec((1, SH), lambda i: (0, 0)),
            pl.BlockSpec((1, QS + SH), lambda i: (0, 0)),
            pl.BlockSpec(memory_space=pltpu.MemorySpace.SMEM),
        ],
        out_specs=[
            pl.BlockSpec((QS, BLK), lambda i: (0, i)),
            pl.BlockSpec((SH, BLK), lambda i: (0, i)),
            pl.BlockSpec((1, 1), lambda i: (0, 0)),
        ],
        out_shape=[
            jax.ShapeDtypeStruct((QS, T + 1), f32),
            jax.ShapeDtypeStruct((SH, T + 1), f32),
            jax.ShapeDtypeStruct((1, 1), f32),
        ],
        scratch_shapes=[pltpu.SMEM((1,), f32), pltpu.VMEM((SH, 1), f32)],
    )(qhT, hsT, q, hnew, sW, sb.astype(f32))

    qn = qnT.T
    hn = jnp.transpose(hnT.reshape(1, SH, T + 1), (2, 0, 1))
    return pred, qn, hn
